# Initial kernel scaffold; baseline (speedup 1.0000x reference)
import jax
import jax.numpy as jnp
from jax.experimental import pallas as pl


def kernel(x, edge_index, edge_attr, lw0, lb0, w10, b10, w20, b20, lw1, lb1, w11, b11, w21, b21):
    raise NotImplementedError("write your pallas kernel here")


# speedup vs baseline: 2.2073x; 2.2073x over previous
import functools

import jax
import jax.numpy as jnp
from jax import lax
from jax.experimental import pallas as pl
from jax.experimental.pallas import tpu as pltpu
from jax.experimental.pallas import tpu_sc as plsc

N, E, D, DE, HID = 10000, 320000, 128, 16, 256

NC, NS = 2, 16
DH = D // NC
EPW = E // NS
CHUNK = 80
NCHUNK = EPW // CHUNK
NPAD = 10240
ROWS_PER_SUB = NPAD // NS


def _edge_lin_body(ea_ref, lw0_ref, lb0_ref, lw1_ref, lb1_ref, e0_ref, e1_ref):
    ea = ea_ref[...]
    e0 = (jnp.dot(ea, lw0_ref[...], preferred_element_type=jnp.float32)
          + lb0_ref[...])
    e1 = (jnp.dot(ea, lw1_ref[...], preferred_element_type=jnp.float32)
          + lb1_ref[...])
    e0_ref[0] = e0[:, :DH]
    e0_ref[1] = e0[:, DH:]
    e1_ref[0] = e1[:, :DH]
    e1_ref[1] = e1[:, DH:]


_BE = 4000


def _edge_lin(edge_attr, lw0, lb0, lw1, lb1):
    return pl.pallas_call(
        _edge_lin_body,
        grid=(E // _BE,),
        in_specs=[
            pl.BlockSpec((_BE, DE), lambda i: (i, 0)),
            pl.BlockSpec((DE, D), lambda i: (0, 0)),
            pl.BlockSpec((1, D), lambda i: (0, 0)),
            pl.BlockSpec((DE, D), lambda i: (0, 0)),
            pl.BlockSpec((1, D), lambda i: (0, 0)),
        ],
        out_specs=[
            pl.BlockSpec((NC, _BE, DH), lambda i: (0, i, 0)),
            pl.BlockSpec((NC, _BE, DH), lambda i: (0, i, 0)),
        ],
        out_shape=[
            jax.ShapeDtypeStruct((NC, E, DH), jnp.float32),
            jax.ShapeDtypeStruct((NC, E, DH), jnp.float32),
        ],
    )(edge_attr, lw0, lb0.reshape(1, D), lw1, lb1.reshape(1, D))


def _sc_agg_body(h_hbm, e_hbm, src_hbm, dst_hbm, out_hbm,
                 src_v, dst_v, hbuf, ebuf, agg, sem_h, sem_e):
    c = lax.axis_index("c")
    s = lax.axis_index("s")

    pltpu.sync_copy(src_hbm.at[c, s], src_v)
    pltpu.sync_copy(dst_hbm.at[s], dst_v)

    zero16 = jnp.zeros((16,), jnp.float32)

    def zrow(i, carry):
        for j in range(DH // 16):
            ebuf[i, pl.ds(j * 16, 16)] = zero16
        return carry

    lax.fori_loop(0, CHUNK, zrow, 0)
    base_n = s * ROWS_PER_SUB
    for r in range(ROWS_PER_SUB // CHUNK):
        pltpu.sync_copy(ebuf, agg.at[pl.ds(base_n + r * CHUNK, CHUNK), :])
    plsc.subcore_barrier()

    edge_base = s * EPW

    def chunk(g, carry):
        off = edge_base + g * CHUNK
        cp_h = pltpu.async_copy(h_hbm.at[src_v.at[g]], hbuf, sem_h)
        cp_e = pltpu.async_copy(e_hbm.at[c, pl.ds(off, CHUNK), :], ebuf, sem_e)
        cp_h.wait()
        cp_e.wait()

        def mrow(i, inner):
            for j in range(DH // 16):
                dsl = pl.ds(j * 16, 16)
                ebuf[i, dsl] = jnp.maximum(hbuf[i, dsl] + ebuf[i, dsl], 0.0)
            return inner

        lax.fori_loop(0, CHUNK, mrow, 0)
        pltpu.sync_copy(ebuf, agg.at[dst_v.at[g]], add=True)
        return carry

    lax.fori_loop(0, NCHUNK, chunk, 0)
    plsc.subcore_barrier()

    pltpu.sync_copy(agg.at[pl.ds(base_n, ROWS_PER_SUB), :],
                    out_hbm.at[c, pl.ds(base_n, ROWS_PER_SUB), :])


_sc_agg = pl.kernel(
    _sc_agg_body,
    out_type=jax.ShapeDtypeStruct((NC, NPAD, DH), jnp.float32),
    mesh=plsc.VectorSubcoreMesh(core_axis_name="c", subcore_axis_name="s"),
    scratch_types=[
        pltpu.VMEM((NCHUNK, CHUNK), jnp.int32),
        pltpu.VMEM((NCHUNK, CHUNK), jnp.int32),
        pltpu.VMEM((CHUNK, DH), jnp.float32),
        pltpu.VMEM((CHUNK, DH), jnp.float32),
        pltpu.VMEM_SHARED((NPAD, DH), jnp.float32),
        pltpu.SemaphoreType.DMA,
        pltpu.SemaphoreType.DMA,
    ],
    compiler_params=pltpu.CompilerParams(use_tc_tiling_on_sc=False),
)


def _node_mlp_body(out_relu, p_ref, h_ref, w1_ref, b1_ref, w2_ref, b2_ref,
                   y_ref):
    node = jnp.concatenate([p_ref[0], p_ref[1]], axis=-1) + h_ref[...]
    hid = jnp.maximum(
        jnp.dot(node, w1_ref[...], preferred_element_type=jnp.float32)
        + b1_ref[...], 0.0)
    y = (jnp.dot(hid, w2_ref[...], preferred_element_type=jnp.float32)
         + b2_ref[...])
    if out_relu:
        y = jnp.maximum(y, 0.0)
    y_ref[...] = y


_BN = 1000


def _node_mlp(parts, h, w1, b1, w2, b2, out_relu):
    return pl.pallas_call(
        functools.partial(_node_mlp_body, out_relu),
        grid=(N // _BN,),
        in_specs=[
            pl.BlockSpec((NC, _BN, DH), lambda i: (0, i, 0)),
            pl.BlockSpec((_BN, D), lambda i: (i, 0)),
            pl.BlockSpec((D, HID), lambda i: (0, 0)),
            pl.BlockSpec((1, HID), lambda i: (0, 0)),
            pl.BlockSpec((HID, D), lambda i: (0, 0)),
            pl.BlockSpec((1, D), lambda i: (0, 0)),
        ],
        out_specs=pl.BlockSpec((_BN, D), lambda i: (i, 0)),
        out_shape=jax.ShapeDtypeStruct((N, D), jnp.float32),
    )(parts, h, w1, b1.reshape(1, HID), w2, b2.reshape(1, D))


def kernel(x, edge_index, edge_attr, lw0, lb0, w10, b10, w20, b20,
           lw1, lb1, w11, b11, w21, b21):
    e0, e1 = _edge_lin(edge_attr, lw0, lb0, lw1, lb1)
    src = edge_index[0]
    src4 = jnp.stack([src * 2, src * 2 + 1]).reshape(NC, NS, NCHUNK, CHUNK)
    dst3 = edge_index[1].reshape(NS, NCHUNK, CHUNK)

    p0 = _sc_agg(x.reshape(NC * N, DH), e0, src4, dst3)
    h1 = _node_mlp(p0, x, w10, b10, w20, b20, out_relu=True)

    p1 = _sc_agg(h1.reshape(NC * N, DH), e1, src4, dst3)
    return _node_mlp(p1, h1, w11, b11, w21, b21, out_relu=False)


# speedup vs baseline: 2.8857x; 1.3074x over previous
import functools

import jax
import jax.numpy as jnp
from jax import lax
from jax.experimental import pallas as pl
from jax.experimental.pallas import tpu as pltpu
from jax.experimental.pallas import tpu_sc as plsc

N, E, D, DE, HID = 10000, 320000, 128, 16, 256

NC, NS = 2, 16
DH = D // NC
EPW = E // NS
CHUNK = 80
NCHUNK = EPW // CHUNK
NPAD = 10240
ROWS_PER_SUB = NPAD // NS


def _edge_lin_body(ea_ref, lw0_ref, lb0_ref, lw1_ref, lb1_ref, e0_ref, e1_ref):
    ea = ea_ref[...]
    e0 = (jnp.dot(ea, lw0_ref[...], preferred_element_type=jnp.float32)
          + lb0_ref[...])
    e1 = (jnp.dot(ea, lw1_ref[...], preferred_element_type=jnp.float32)
          + lb1_ref[...])
    e0_ref[0] = e0[:, :DH]
    e0_ref[1] = e0[:, DH:]
    e1_ref[0] = e1[:, :DH]
    e1_ref[1] = e1[:, DH:]


_BE = 4000


def _edge_lin(edge_attr, lw0, lb0, lw1, lb1):
    return pl.pallas_call(
        _edge_lin_body,
        grid=(E // _BE,),
        in_specs=[
            pl.BlockSpec((_BE, DE), lambda i: (i, 0)),
            pl.BlockSpec((DE, D), lambda i: (0, 0)),
            pl.BlockSpec((1, D), lambda i: (0, 0)),
            pl.BlockSpec((DE, D), lambda i: (0, 0)),
            pl.BlockSpec((1, D), lambda i: (0, 0)),
        ],
        out_specs=[
            pl.BlockSpec((NC, _BE, DH), lambda i: (0, i, 0)),
            pl.BlockSpec((NC, _BE, DH), lambda i: (0, i, 0)),
        ],
        out_shape=[
            jax.ShapeDtypeStruct((NC, E, DH), jnp.float32),
            jax.ShapeDtypeStruct((NC, E, DH), jnp.float32),
        ],
    )(edge_attr, lw0, lb0.reshape(1, D), lw1, lb1.reshape(1, D))


def _sc_agg_body(h_hbm, e_hbm, src_hbm, dst_hbm, out_hbm,
                 src_v, dst_v, hbuf0, ebuf0, hbuf1, ebuf1, agg,
                 sem_h0, sem_e0, sem_h1, sem_e1):
    c = lax.axis_index("c")
    s = lax.axis_index("s")

    pltpu.sync_copy(src_hbm.at[c, s], src_v)
    pltpu.sync_copy(dst_hbm.at[s], dst_v)

    zero16 = jnp.zeros((16,), jnp.float32)

    def zrow(i, carry):
        for j in range(DH // 16):
            ebuf0[i, pl.ds(j * 16, 16)] = zero16
        return carry

    lax.fori_loop(0, CHUNK, zrow, 0)
    base_n = s * ROWS_PER_SUB
    for r in range(ROWS_PER_SUB // CHUNK):
        pltpu.sync_copy(ebuf0, agg.at[pl.ds(base_n + r * CHUNK, CHUNK), :])
    plsc.subcore_barrier()

    edge_base = s * EPW
    bufs = ((hbuf0, ebuf0, sem_h0, sem_e0), (hbuf1, ebuf1, sem_h1, sem_e1))

    def start(g, slot):
        hb, eb, sh, se = bufs[slot]
        off = edge_base + g * CHUNK
        pltpu.async_copy(h_hbm.at[src_v.at[g]], hb, sh)
        pltpu.async_copy(e_hbm.at[c, pl.ds(off, CHUNK), :], eb, se)

    def finish(g, slot):
        hb, eb, sh, se = bufs[slot]
        pltpu.make_async_copy(h_hbm.at[src_v.at[0]], hb, sh).wait()
        pltpu.make_async_copy(e_hbm.at[c, pl.ds(0, CHUNK), :], eb, se).wait()

        def mrow(i, inner):
            for j in range(DH // 16):
                dsl = pl.ds(j * 16, 16)
                eb[i, dsl] = jnp.maximum(hb[i, dsl] + eb[i, dsl], 0.0)
            return inner

        lax.fori_loop(0, CHUNK, mrow, 0)
        pltpu.sync_copy(eb, agg.at[dst_v.at[g]], add=True)

    start(0, 0)
    start(1, 1)

    def chunk2(t, carry):
        g = t * 2
        finish(g, 0)

        @pl.when(g + 2 < NCHUNK)
        def _():
            start(g + 2, 0)

        finish(g + 1, 1)

        @pl.when(g + 3 < NCHUNK)
        def _():
            start(g + 3, 1)

        return carry

    lax.fori_loop(0, NCHUNK // 2, chunk2, 0)
    plsc.subcore_barrier()

    pltpu.sync_copy(agg.at[pl.ds(base_n, ROWS_PER_SUB), :],
                    out_hbm.at[c, pl.ds(base_n, ROWS_PER_SUB), :])


_sc_agg = pl.kernel(
    _sc_agg_body,
    out_type=jax.ShapeDtypeStruct((NC, NPAD, DH), jnp.float32),
    mesh=plsc.VectorSubcoreMesh(core_axis_name="c", subcore_axis_name="s"),
    scratch_types=[
        pltpu.VMEM((NCHUNK, CHUNK), jnp.int32),
        pltpu.VMEM((NCHUNK, CHUNK), jnp.int32),
        pltpu.VMEM((CHUNK, DH), jnp.float32),
        pltpu.VMEM((CHUNK, DH), jnp.float32),
        pltpu.VMEM((CHUNK, DH), jnp.float32),
        pltpu.VMEM((CHUNK, DH), jnp.float32),
        pltpu.VMEM_SHARED((NPAD, DH), jnp.float32),
        pltpu.SemaphoreType.DMA,
        pltpu.SemaphoreType.DMA,
        pltpu.SemaphoreType.DMA,
        pltpu.SemaphoreType.DMA,
    ],
    compiler_params=pltpu.CompilerParams(use_tc_tiling_on_sc=False),
)


def _node_mlp_body(out_relu, p_ref, h_ref, w1_ref, b1_ref, w2_ref, b2_ref,
                   y_ref):
    node = jnp.concatenate([p_ref[0], p_ref[1]], axis=-1) + h_ref[...]
    hid = jnp.maximum(
        jnp.dot(node, w1_ref[...], preferred_element_type=jnp.float32)
        + b1_ref[...], 0.0)
    y = (jnp.dot(hid, w2_ref[...], preferred_element_type=jnp.float32)
         + b2_ref[...])
    if out_relu:
        y = jnp.maximum(y, 0.0)
    y_ref[...] = y


_BN = 1000


def _node_mlp(parts, h, w1, b1, w2, b2, out_relu):
    return pl.pallas_call(
        functools.partial(_node_mlp_body, out_relu),
        grid=(N // _BN,),
        in_specs=[
            pl.BlockSpec((NC, _BN, DH), lambda i: (0, i, 0)),
            pl.BlockSpec((_BN, D), lambda i: (i, 0)),
            pl.BlockSpec((D, HID), lambda i: (0, 0)),
            pl.BlockSpec((1, HID), lambda i: (0, 0)),
            pl.BlockSpec((HID, D), lambda i: (0, 0)),
            pl.BlockSpec((1, D), lambda i: (0, 0)),
        ],
        out_specs=pl.BlockSpec((_BN, D), lambda i: (i, 0)),
        out_shape=jax.ShapeDtypeStruct((N, D), jnp.float32),
    )(parts, h, w1, b1.reshape(1, HID), w2, b2.reshape(1, D))


def kernel(x, edge_index, edge_attr, lw0, lb0, w10, b10, w20, b20,
           lw1, lb1, w11, b11, w21, b21):
    e0, e1 = _edge_lin(edge_attr, lw0, lb0, lw1, lb1)
    src = edge_index[0]
    src4 = jnp.stack([src * 2, src * 2 + 1]).reshape(NC, NS, NCHUNK, CHUNK)
    dst3 = edge_index[1].reshape(NS, NCHUNK, CHUNK)

    p0 = _sc_agg(x.reshape(NC * N, DH), e0, src4, dst3)
    h1 = _node_mlp(p0, x, w10, b10, w20, b20, out_relu=True)

    p1 = _sc_agg(h1.reshape(NC * N, DH), e1, src4, dst3)
    return _node_mlp(p1, h1, w11, b11, w21, b21, out_relu=False)


# speedup vs baseline: 4.2819x; 1.4838x over previous
import functools

import jax
import jax.numpy as jnp
from jax import lax
from jax.experimental import pallas as pl
from jax.experimental.pallas import tpu as pltpu
from jax.experimental.pallas import tpu_sc as plsc

N, E, D, DE, HID = 10000, 320000, 128, 16, 256

NC, NS = 2, 16
DH = D // NC
EPW = E // NS
CHUNK = 80
NCHUNK = EPW // CHUNK
NPAD = 10240
ROWS_PER_SUB = NPAD // NS


def _edge_lin_body(ea_ref, w_ref, b_ref, out_ref):
    ea = ea_ref[...]
    for c in range(NC):
        out_ref[c] = (
            jnp.dot(ea, w_ref[c], preferred_element_type=jnp.float32)
            + b_ref[c]
        )


_BE2 = 2000


def _edge_lin(edge_attr_pair, lw, lb):
    z = jnp.zeros((DE, DH), jnp.float32)
    wp = jnp.stack([
        jnp.block([[lw[:, :DH], z], [z, lw[:, :DH]]]),
        jnp.block([[lw[:, DH:], z], [z, lw[:, DH:]]]),
    ])
    bp = jnp.stack([
        jnp.concatenate([lb[:DH], lb[:DH]]).reshape(1, D),
        jnp.concatenate([lb[DH:], lb[DH:]]).reshape(1, D),
    ])
    return pl.pallas_call(
        _edge_lin_body,
        grid=(E // 2 // _BE2,),
        in_specs=[
            pl.BlockSpec((_BE2, 2 * DE), lambda i: (i, 0)),
            pl.BlockSpec((NC, 2 * DE, D), lambda i: (0, 0, 0)),
            pl.BlockSpec((NC, 1, D), lambda i: (0, 0, 0)),
        ],
        out_specs=pl.BlockSpec((NC, _BE2, D), lambda i: (0, i, 0)),
        out_shape=jax.ShapeDtypeStruct((NC, E // 2, D), jnp.float32),
    )(edge_attr_pair, wp, bp)


def _sc_agg_body(h_hbm, e_hbm, src_hbm, dst_hbm, out_hbm,
                 src_v, dst_v, idx0, idx1, hbuf0, ebuf0, mbuf0,
                 hbuf1, ebuf1, mbuf1, agg,
                 sem_h0, sem_e0, sem_h1, sem_e1):
    c = lax.axis_index("c")
    s = lax.axis_index("s")

    pltpu.sync_copy(src_hbm.at[s], src_v)
    pltpu.sync_copy(dst_hbm.at[s], dst_v)

    zero16 = jnp.zeros((16,), jnp.float32)

    def zrow(i, carry):
        for j in range(DH // 16):
            mbuf0[i, pl.ds(j * 16, 16)] = zero16
        return carry

    lax.fori_loop(0, CHUNK, zrow, 0)
    base_n = s * ROWS_PER_SUB
    for r in range(ROWS_PER_SUB // CHUNK):
        pltpu.sync_copy(mbuf0, agg.at[pl.ds(base_n + r * CHUNK, CHUNK), :])
    plsc.subcore_barrier()

    edge_base = s * EPW
    bufs = ((idx0, hbuf0, ebuf0, mbuf0, sem_h0, sem_e0),
            (idx1, hbuf1, ebuf1, mbuf1, sem_h1, sem_e1))

    def start(g, slot):
        idx, hb, eb, _, sh, se = bufs[slot]

        def irow(k, carry):
            dsl = pl.ds(k * 16, 16)
            idx[dsl] = src_v[g, dsl] * 2 + c
            return carry

        lax.fori_loop(0, CHUNK // 16, irow, 0)
        pltpu.async_copy(h_hbm.at[idx], hb, sh)
        off2 = (edge_base + g * CHUNK) // 2
        pltpu.async_copy(e_hbm.at[c, pl.ds(off2, CHUNK // 2), :], eb, se)

    def finish(g, slot):
        idx, hb, eb, mb, sh, se = bufs[slot]
        pltpu.make_async_copy(h_hbm.at[idx], hb, sh).wait()
        pltpu.make_async_copy(e_hbm.at[c, pl.ds(0, CHUNK // 2), :], eb,
                              se).wait()

        def mrow(i, inner):
            for j in range(DH // 16):
                dsl = pl.ds(j * 16, 16)
                mb[2 * i, dsl] = jnp.maximum(
                    hb[2 * i, dsl] + eb[i, dsl], 0.0)
                mb[2 * i + 1, dsl] = jnp.maximum(
                    hb[2 * i + 1, dsl] + eb[i, pl.ds(DH + j * 16, 16)], 0.0)
            return inner

        lax.fori_loop(0, CHUNK // 2, mrow, 0)
        pltpu.sync_copy(mb, agg.at[dst_v.at[g]], add=True)

    start(0, 0)
    start(1, 1)

    def chunk2(t, carry):
        g = t * 2
        finish(g, 0)

        @pl.when(g + 2 < NCHUNK)
        def _():
            start(g + 2, 0)

        finish(g + 1, 1)

        @pl.when(g + 3 < NCHUNK)
        def _():
            start(g + 3, 1)

        return carry

    lax.fori_loop(0, NCHUNK // 2, chunk2, 0)
    plsc.subcore_barrier()

    pltpu.sync_copy(agg.at[pl.ds(base_n, ROWS_PER_SUB), :],
                    out_hbm.at[c, pl.ds(base_n, ROWS_PER_SUB), :])


_sc_agg = pl.kernel(
    _sc_agg_body,
    out_type=jax.ShapeDtypeStruct((NC, NPAD, DH), jnp.float32),
    mesh=plsc.VectorSubcoreMesh(core_axis_name="c", subcore_axis_name="s"),
    scratch_types=[
        pltpu.VMEM((NCHUNK, CHUNK), jnp.int32),
        pltpu.VMEM((NCHUNK, CHUNK), jnp.int32),
        pltpu.VMEM((CHUNK,), jnp.int32),
        pltpu.VMEM((CHUNK,), jnp.int32),
        pltpu.VMEM((CHUNK, DH), jnp.float32),
        pltpu.VMEM((CHUNK // 2, D), jnp.float32),
        pltpu.VMEM((CHUNK, DH), jnp.float32),
        pltpu.VMEM((CHUNK, DH), jnp.float32),
        pltpu.VMEM((CHUNK // 2, D), jnp.float32),
        pltpu.VMEM((CHUNK, DH), jnp.float32),
        pltpu.VMEM_SHARED((NPAD, DH), jnp.float32),
        pltpu.SemaphoreType.DMA,
        pltpu.SemaphoreType.DMA,
        pltpu.SemaphoreType.DMA,
        pltpu.SemaphoreType.DMA,
    ],
    compiler_params=pltpu.CompilerParams(use_tc_tiling_on_sc=False),
)


def _node_mlp_body(out_relu, p_ref, h_ref, w1_ref, b1_ref, w2_ref, b2_ref,
                   y_ref):
    node = jnp.concatenate([p_ref[0], p_ref[1]], axis=-1) + h_ref[...]
    hid = jnp.maximum(
        jnp.dot(node, w1_ref[...], preferred_element_type=jnp.float32)
        + b1_ref[...], 0.0)
    y = (jnp.dot(hid, w2_ref[...], preferred_element_type=jnp.float32)
         + b2_ref[...])
    if out_relu:
        y = jnp.maximum(y, 0.0)
    y_ref[...] = y


_BN = 1000


def _node_mlp(parts, h, w1, b1, w2, b2, out_relu):
    return pl.pallas_call(
        functools.partial(_node_mlp_body, out_relu),
        grid=(N // _BN,),
        in_specs=[
            pl.BlockSpec((NC, _BN, DH), lambda i: (0, i, 0)),
            pl.BlockSpec((_BN, D), lambda i: (i, 0)),
            pl.BlockSpec((D, HID), lambda i: (0, 0)),
            pl.BlockSpec((1, HID), lambda i: (0, 0)),
            pl.BlockSpec((HID, D), lambda i: (0, 0)),
            pl.BlockSpec((1, D), lambda i: (0, 0)),
        ],
        out_specs=pl.BlockSpec((_BN, D), lambda i: (i, 0)),
        out_shape=jax.ShapeDtypeStruct((N, D), jnp.float32),
    )(parts, h, w1, b1.reshape(1, HID), w2, b2.reshape(1, D))


def kernel(x, edge_index, edge_attr, lw0, lb0, w10, b10, w20, b20,
           lw1, lb1, w11, b11, w21, b21):
    ea_pair = edge_attr.reshape(E // 2, 2 * DE)
    e0 = _edge_lin(ea_pair, lw0, lb0)
    e1 = _edge_lin(ea_pair, lw1, lb1)
    src3 = edge_index[0].reshape(NS, NCHUNK, CHUNK)
    dst3 = edge_index[1].reshape(NS, NCHUNK, CHUNK)

    p0 = _sc_agg(x.reshape(NC * N, DH), e0, src3, dst3)
    h1 = _node_mlp(p0, x, w10, b10, w20, b20, out_relu=True)

    p1 = _sc_agg(h1.reshape(NC * N, DH), e1, src3, dst3)
    return _node_mlp(p1, h1, w11, b11, w21, b21, out_relu=False)
